# baseline (speedup 1.0000x reference)
import functools
import math

import jax
import jax.numpy as jnp
from jax import lax
from jax.experimental import pallas as pl
from jax.experimental.pallas import tpu as pltpu

K = 16
R = 256

SC_IMPL = "sc"


def _select_kernel(x_ref, idx_ref):
    b = pl.program_id(0)
    r = pl.program_id(1)
    xs = x_ref[0]
    n_nodes = xs.shape[0]
    xr = x_ref[0, pl.ds(r * R, R), :]
    sq_full = jnp.sum(xs * xs, axis=1)
    sq_r = jnp.sum(xr * xr, axis=1)
    cross = jax.lax.dot_general(
        xr, xs, (((1,), (1,)), ((), ())),
        preferred_element_type=jnp.float32)
    d2 = sq_r[:, None] - 2.0 * cross + sq_full[None, :]
    rows = jax.lax.broadcasted_iota(jnp.int32, (R, n_nodes), 0) + r * R
    cols = jax.lax.broadcasted_iota(jnp.int32, (R, n_nodes), 1)
    d2 = jnp.where(rows == cols, d2 + 1e9, d2)

    w = d2
    idx_cols = []
    for _ in range(K):
        m = jnp.min(w, axis=1, keepdims=True)
        eq = w == m
        idxk = jnp.min(jnp.where(eq, cols, n_nodes), axis=1,
                       keepdims=True)
        idx_cols.append(idxk)
        w = jnp.where(eq, jnp.inf, w)
    idx_ref[0] = jnp.concatenate(idx_cols, axis=1) + b * n_nodes


def topk_indices(x):
    B, N, D = x.shape
    nb = N // R
    return pl.pallas_call(
        _select_kernel,
        grid=(B, nb),
        in_specs=[pl.BlockSpec((1, N, D), lambda b, r: (b, 0, 0))],
        out_specs=pl.BlockSpec((1, R, K), lambda b, r: (b, r, 0)),
        out_shape=jax.ShapeDtypeStruct((B, N, K), jnp.int32),
    )(x)


def gather_sum_jax(xf, idx_flat):
    BN, D = xf.shape
    return jnp.sum(xf[idx_flat.reshape(BN, K)], axis=1)


def sc_gather_sum(xf, idx_flat):
    from jax.experimental.pallas import tpu_sc as plsc

    BN, D = xf.shape
    info = plsc.get_sparse_core_info()
    NW = info.num_cores * info.num_subcores
    nodes_per_w = BN // NW
    G = 8
    chunks = nodes_per_w // G
    mesh = plsc.VectorSubcoreMesh(core_axis_name="c", subcore_axis_name="s")

    @functools.partial(
        pl.kernel, mesh=mesh,
        out_type=jax.ShapeDtypeStruct((BN, D), jnp.float32),
        scratch_types=[
            pltpu.VMEM((G * K,), jnp.int32),
            pltpu.VMEM((G * K, D), jnp.float32),
            pltpu.VMEM((G, D), jnp.float32),
            pltpu.SemaphoreType.DMA,
        ],
    )
    def k(xf_hbm, idx_hbm, out_hbm, idx_v, rows_v, out_v, sem):
        wid = lax.axis_index("s") * info.num_cores + lax.axis_index("c")
        base = wid * nodes_per_w

        def body(g, carry):
            nb = base + g * G
            pltpu.sync_copy(idx_hbm.at[pl.ds(nb * K, G * K)], idx_v)
            pltpu.async_copy(xf_hbm.at[idx_v], rows_v, sem).wait()
            for n in range(G):
                for c in range(D // 16):
                    acc = rows_v[n * K, pl.ds(c * 16, 16)]
                    for r2 in range(1, K):
                        acc = acc + rows_v[n * K + r2, pl.ds(c * 16, 16)]
                    out_v[n, pl.ds(c * 16, 16)] = acc
            pltpu.sync_copy(out_v, out_hbm.at[pl.ds(nb, G)])
            return carry

        lax.fori_loop(0, chunks, body, 0)

    return k(xf, idx_flat)


def _dense_kernel(x_ref, nbr_ref, t_ref, c_ref, wmsg_ref, bmsg_ref,
                  wtime_ref, wctx_ref, wout_ref, bout_ref, out_ref):
    half = wtime_ref.shape[0] // 2
    freq_i = jax.lax.broadcasted_iota(jnp.int32, (1, half), 1).astype(
        jnp.float32)
    freqs = jnp.exp(freq_i * (-math.log(10000.0) / half))
    args = t_ref[0] * freqs
    temb = jnp.concatenate([jnp.cos(args), jnp.sin(args)], axis=-1)
    cond = (jnp.dot(temb, wtime_ref[...], preferred_element_type=jnp.float32)
            + jnp.dot(c_ref[0], wctx_ref[...],
                      preferred_element_type=jnp.float32))
    xr = x_ref[0]
    agg = (jnp.dot(nbr_ref[0], wmsg_ref[...],
                   preferred_element_type=jnp.float32)
           + float(K) * bmsg_ref[...][None, :])
    h = jnp.maximum(xr + agg + cond, 0.0)
    out_ref[0] = (jnp.dot(h, wout_ref[...], preferred_element_type=jnp.float32)
                  + bout_ref[...][None, :])


def dense_out(x, nbr, t, c_vector, W_msg, b_msg, W_time, W_ctx, W_out, b_out):
    B, N, D = x.shape
    CTX = c_vector.shape[1]
    nb = N // R
    return pl.pallas_call(
        _dense_kernel,
        grid=(B, nb),
        in_specs=[
            pl.BlockSpec((1, R, D), lambda b, r: (b, r, 0)),
            pl.BlockSpec((1, R, D), lambda b, r: (b, r, 0)),
            pl.BlockSpec((1, 1, 1), lambda b, r: (b, 0, 0)),
            pl.BlockSpec((1, 1, CTX), lambda b, r: (b, 0, 0)),
            pl.BlockSpec((D, D), lambda b, r: (0, 0)),
            pl.BlockSpec((D,), lambda b, r: (0,)),
            pl.BlockSpec((D, D), lambda b, r: (0, 0)),
            pl.BlockSpec((CTX, D), lambda b, r: (0, 0)),
            pl.BlockSpec((D, D), lambda b, r: (0, 0)),
            pl.BlockSpec((D,), lambda b, r: (0,)),
        ],
        out_specs=pl.BlockSpec((1, R, D), lambda b, r: (b, r, 0)),
        out_shape=jax.ShapeDtypeStruct((B, N, D), jnp.float32),
    )(x, nbr, t.reshape(B, 1, 1), c_vector.reshape(B, 1, CTX), W_msg, b_msg,
      W_time, W_ctx, W_out, b_out)


def kernel(x, t, c_vector, W_msg, b_msg, W_time, W_ctx, W_out, b_out):
    B, N, D = x.shape
    idx = topk_indices(x)
    xf = x.reshape(B * N, D)
    idx_flat = idx.reshape(B * N * K)
    if SC_IMPL == "jax":
        nbr = gather_sum_jax(xf, idx_flat)
    else:
        nbr = sc_gather_sum(xf, idx_flat)
    return dense_out(x, nbr.reshape(B, N, D), t, c_vector, W_msg, b_msg,
                     W_time, W_ctx, W_out, b_out)


# speedup vs baseline: 1.8773x; 1.8773x over previous
import functools
import math

import jax
import jax.numpy as jnp
from jax import lax
from jax.experimental import pallas as pl
from jax.experimental.pallas import tpu as pltpu

K = 16
R = 256
C = 128


def _fused_kernel(x_ref, t_ref, c_ref, wmsg_ref, bmsg_ref, wtime_ref,
                  wctx_ref, wout_ref, bout_ref, out_ref, sq_ref):
    r = pl.program_id(1)
    xs = x_ref[0]
    n_nodes = xs.shape[0]

    @pl.when(r == 0)
    def _():
        sq_ref[...] = jnp.sum(xs * xs, axis=1)[None, :]

    half = wtime_ref.shape[0] // 2
    freq_i = jax.lax.broadcasted_iota(jnp.int32, (1, half), 1).astype(
        jnp.float32)
    freqs = jnp.exp(freq_i * (-math.log(10000.0) / half))
    args = t_ref[0] * freqs
    temb = jnp.concatenate([jnp.cos(args), jnp.sin(args)], axis=-1)
    cond = (jnp.dot(temb, wtime_ref[...], preferred_element_type=jnp.float32)
            + jnp.dot(c_ref[0], wctx_ref[...],
                      preferred_element_type=jnp.float32))

    xr = x_ref[0, pl.ds(r * R, R), :]
    cross = jax.lax.dot_general(
        xr, xs, (((1,), (1,)), ((), ())),
        preferred_element_type=jnp.float32)
    s = sq_ref[...] - 2.0 * cross
    rows = jax.lax.broadcasted_iota(jnp.int32, (R, n_nodes), 0) + r * R
    cols = jax.lax.broadcasted_iota(jnp.int32, (R, n_nodes), 1)
    s = jnp.where(rows == cols, s + 1e9, s)

    nchunk = n_nodes // C
    chunks = [s[:, g * C:(g + 1) * C] for g in range(nchunk)]
    red1 = functools.reduce(jnp.minimum, chunks)
    red2 = functools.reduce(
        jnp.minimum,
        [jnp.where(ch == red1, jnp.inf, ch) for ch in chunks])
    cand = jnp.concatenate([red1, red2], axis=1)
    tau = None
    for _ in range(K):
        tau = jnp.min(cand, axis=1, keepdims=True)
        cand = jnp.where(cand == tau, jnp.inf, cand)
    selmask = jnp.where(s <= tau, 1.0, 0.0)
    cnt = jnp.sum(selmask, axis=1, keepdims=True)
    nbad = jnp.sum(jnp.where(cnt == float(K), 0.0, 1.0))

    def _exact_thr():
        w = s
        m = None
        for _ in range(K):
            m = jnp.min(w, axis=1, keepdims=True)
            w = jnp.where(w == m, jnp.inf, w)
        return m

    thr = lax.cond(nbad > 0.0, _exact_thr, lambda: tau)
    mask = jnp.where(s <= thr, 1.0, 0.0)

    nbr = jnp.dot(mask, xs, preferred_element_type=jnp.float32)
    agg = (jnp.dot(nbr, wmsg_ref[...], preferred_element_type=jnp.float32)
           + float(K) * bmsg_ref[...][None, :])
    h = jnp.maximum(xr + agg + cond, 0.0)
    out_ref[0] = (jnp.dot(h, wout_ref[...], preferred_element_type=jnp.float32)
                  + bout_ref[...][None, :])


def kernel(x, t, c_vector, W_msg, b_msg, W_time, W_ctx, W_out, b_out):
    B, N, D = x.shape
    CTX = c_vector.shape[1]
    nb = N // R
    grid = (B, nb)
    out = pl.pallas_call(
        _fused_kernel,
        grid=grid,
        in_specs=[
            pl.BlockSpec((1, N, D), lambda b, r: (b, 0, 0)),
            pl.BlockSpec((1, 1, 1), lambda b, r: (b, 0, 0)),
            pl.BlockSpec((1, 1, CTX), lambda b, r: (b, 0, 0)),
            pl.BlockSpec((D, D), lambda b, r: (0, 0)),
            pl.BlockSpec((D,), lambda b, r: (0,)),
            pl.BlockSpec((D, D), lambda b, r: (0, 0)),
            pl.BlockSpec((CTX, D), lambda b, r: (0, 0)),
            pl.BlockSpec((D, D), lambda b, r: (0, 0)),
            pl.BlockSpec((D,), lambda b, r: (0,)),
        ],
        out_specs=pl.BlockSpec((1, R, D), lambda b, r: (b, r, 0)),
        out_shape=jax.ShapeDtypeStruct((B, N, D), jnp.float32),
        scratch_shapes=[pltpu.VMEM((1, N), jnp.float32)],
    )(x, t.reshape(B, 1, 1), c_vector.reshape(B, 1, CTX), W_msg, b_msg,
      W_time, W_ctx, W_out, b_out)
    return out


# speedup vs baseline: 2.6572x; 1.4154x over previous
import functools
import math

import jax
import jax.numpy as jnp
from jax import lax
from jax.experimental import pallas as pl
from jax.experimental.pallas import tpu as pltpu

K = 16
R = 256
C = 128


def _fused_kernel(x_ref, t_ref, c_ref, wmsg_ref, bmsg_ref, wtime_ref,
                  wctx_ref, wout_ref, bout_ref, out_ref, sq_ref):
    r = pl.program_id(1)
    xs = x_ref[0]
    n_nodes = xs.shape[0]

    @pl.when(r == 0)
    def _():
        sq_ref[...] = jnp.sum(xs * xs, axis=1)[None, :]

    half = wtime_ref.shape[0] // 2
    freq_i = jax.lax.broadcasted_iota(jnp.int32, (1, half), 1).astype(
        jnp.float32)
    freqs = jnp.exp(freq_i * (-math.log(10000.0) / half))
    args = t_ref[0] * freqs
    temb = jnp.concatenate([jnp.cos(args), jnp.sin(args)], axis=-1)
    cond = (jnp.dot(temb, wtime_ref[...], preferred_element_type=jnp.float32)
            + jnp.dot(c_ref[0], wctx_ref[...],
                      preferred_element_type=jnp.float32))

    xr = x_ref[0, pl.ds(r * R, R), :]
    cross = jax.lax.dot_general(
        xr, xs, (((1,), (1,)), ((), ())),
        preferred_element_type=jnp.float32)
    s = sq_ref[...] - 2.0 * cross
    rows = jax.lax.broadcasted_iota(jnp.int32, (R, n_nodes), 0) + r * R
    cols = jax.lax.broadcasted_iota(jnp.int32, (R, n_nodes), 1)
    s = jnp.where(rows == cols, s + 1e9, s)

    LEVELS = 4
    nchunk = n_nodes // C
    chunks = [s[:, g * C:(g + 1) * C] for g in range(nchunk)]
    reds = []
    for _ in range(LEVELS):
        red = functools.reduce(jnp.minimum, chunks)
        reds.append(red)
        chunks = [jnp.where(ch == red, jnp.inf, ch) for ch in chunks]
    cand = jnp.concatenate(reds, axis=1)
    tau = None
    for _ in range(K):
        tau = jnp.min(cand, axis=1, keepdims=True)
        cand = jnp.where(cand == tau, jnp.inf, cand)
    selmask = jnp.where(s <= tau, 1.0, 0.0)
    cnt = jnp.sum(selmask, axis=1, keepdims=True)
    nbad = jnp.sum(jnp.where(cnt == float(K), 0.0, 1.0))

    def _exact_mask():
        w = s
        m = None
        for _ in range(K):
            m = jnp.min(w, axis=1, keepdims=True)
            w = jnp.where(w == m, jnp.inf, w)
        return jnp.where(s <= m, 1.0, 0.0)

    mask = lax.cond(nbad > 0.0, _exact_mask, lambda: selmask)

    nbr = jnp.dot(mask, xs, preferred_element_type=jnp.float32)
    agg = (jnp.dot(nbr, wmsg_ref[...], preferred_element_type=jnp.float32)
           + float(K) * bmsg_ref[...][None, :])
    h = jnp.maximum(xr + agg + cond, 0.0)
    out_ref[0] = (jnp.dot(h, wout_ref[...], preferred_element_type=jnp.float32)
                  + bout_ref[...][None, :])


def kernel(x, t, c_vector, W_msg, b_msg, W_time, W_ctx, W_out, b_out):
    B, N, D = x.shape
    CTX = c_vector.shape[1]
    nb = N // R
    grid = (B, nb)
    out = pl.pallas_call(
        _fused_kernel,
        grid=grid,
        in_specs=[
            pl.BlockSpec((1, N, D), lambda b, r: (b, 0, 0)),
            pl.BlockSpec((1, 1, 1), lambda b, r: (b, 0, 0)),
            pl.BlockSpec((1, 1, CTX), lambda b, r: (b, 0, 0)),
            pl.BlockSpec((D, D), lambda b, r: (0, 0)),
            pl.BlockSpec((D,), lambda b, r: (0,)),
            pl.BlockSpec((D, D), lambda b, r: (0, 0)),
            pl.BlockSpec((CTX, D), lambda b, r: (0, 0)),
            pl.BlockSpec((D, D), lambda b, r: (0, 0)),
            pl.BlockSpec((D,), lambda b, r: (0,)),
        ],
        out_specs=pl.BlockSpec((1, R, D), lambda b, r: (b, r, 0)),
        out_shape=jax.ShapeDtypeStruct((B, N, D), jnp.float32),
        scratch_shapes=[pltpu.VMEM((1, N), jnp.float32)],
    )(x, t.reshape(B, 1, 1), c_vector.reshape(B, 1, CTX), W_msg, b_msg,
      W_time, W_ctx, W_out, b_out)
    return out


# speedup vs baseline: 2.8527x; 1.0736x over previous
import functools
import math

import jax
import jax.numpy as jnp
from jax import lax
from jax.experimental import pallas as pl
from jax.experimental.pallas import tpu as pltpu

K = 16
R = 256
C = 128


def _fused_kernel(x_ref, t_ref, c_ref, wmsg_ref, bmsg_ref, wtime_ref,
                  wctx_ref, wout_ref, bout_ref, out_ref, sq_ref):
    r = pl.program_id(1)
    xs = x_ref[0]
    n_nodes = xs.shape[0]

    @pl.when(r == 0)
    def _():
        sq_ref[...] = jnp.sum(xs * xs, axis=1)[None, :]

    half = wtime_ref.shape[0] // 2
    freq_i = jax.lax.broadcasted_iota(jnp.int32, (1, half), 1).astype(
        jnp.float32)
    freqs = jnp.exp(freq_i * (-math.log(10000.0) / half))
    args = t_ref[0] * freqs
    temb = jnp.concatenate([jnp.cos(args), jnp.sin(args)], axis=-1)
    cond = (jnp.dot(temb, wtime_ref[...], preferred_element_type=jnp.float32)
            + jnp.dot(c_ref[0], wctx_ref[...],
                      preferred_element_type=jnp.float32))

    xr = x_ref[0, pl.ds(r * R, R), :]
    cross = jax.lax.dot_general(
        xr, xs, (((1,), (1,)), ((), ())),
        preferred_element_type=jnp.float32)
    s = sq_ref[...] - 2.0 * cross
    rows = jax.lax.broadcasted_iota(jnp.int32, (R, n_nodes), 0) + r * R
    cols = jax.lax.broadcasted_iota(jnp.int32, (R, n_nodes), 1)
    s = jnp.where(rows == cols, s + 1e9, s)

    nchunk = n_nodes // C
    inf = jnp.full((R, C), jnp.inf, dtype=jnp.float32)
    m1, m2, m3, m4 = inf, inf, inf, inf
    for g in range(nchunk):
        v = s[:, g * C:(g + 1) * C]
        b1 = jnp.maximum(m1, v)
        m1 = jnp.minimum(m1, v)
        b2 = jnp.maximum(m2, b1)
        m2 = jnp.minimum(m2, b1)
        b3 = jnp.maximum(m3, b2)
        m3 = jnp.minimum(m3, b2)
        m4 = jnp.minimum(m4, b3)
    tau = None
    for _ in range(K):
        tau = jnp.min(m1, axis=1, keepdims=True)
        eq = m1 == tau
        m1 = jnp.where(eq, m2, m1)
        m2 = jnp.where(eq, m3, m2)
        m3 = jnp.where(eq, m4, m3)
        m4 = jnp.where(eq, jnp.inf, m4)
    selmask = jnp.where(s <= tau, 1.0, 0.0)
    cnt = jnp.sum(selmask, axis=1, keepdims=True)
    nbad = jnp.sum(jnp.where(cnt == float(K), 0.0, 1.0))

    def _exact_mask():
        w = s
        m = None
        for _ in range(K):
            m = jnp.min(w, axis=1, keepdims=True)
            w = jnp.where(w == m, jnp.inf, w)
        return jnp.where(s <= m, 1.0, 0.0)

    mask = lax.cond(nbad > 0.0, _exact_mask, lambda: selmask)

    nbr = jnp.dot(mask, xs, preferred_element_type=jnp.float32)
    agg = (jnp.dot(nbr, wmsg_ref[...], preferred_element_type=jnp.float32)
           + float(K) * bmsg_ref[...][None, :])
    h = jnp.maximum(xr + agg + cond, 0.0)
    out_ref[0] = (jnp.dot(h, wout_ref[...], preferred_element_type=jnp.float32)
                  + bout_ref[...][None, :])


def kernel(x, t, c_vector, W_msg, b_msg, W_time, W_ctx, W_out, b_out):
    B, N, D = x.shape
    CTX = c_vector.shape[1]
    nb = N // R
    grid = (B, nb)
    out = pl.pallas_call(
        _fused_kernel,
        grid=grid,
        in_specs=[
            pl.BlockSpec((1, N, D), lambda b, r: (b, 0, 0)),
            pl.BlockSpec((1, 1, 1), lambda b, r: (b, 0, 0)),
            pl.BlockSpec((1, 1, CTX), lambda b, r: (b, 0, 0)),
            pl.BlockSpec((D, D), lambda b, r: (0, 0)),
            pl.BlockSpec((D,), lambda b, r: (0,)),
            pl.BlockSpec((D, D), lambda b, r: (0, 0)),
            pl.BlockSpec((CTX, D), lambda b, r: (0, 0)),
            pl.BlockSpec((D, D), lambda b, r: (0, 0)),
            pl.BlockSpec((D,), lambda b, r: (0,)),
        ],
        out_specs=pl.BlockSpec((1, R, D), lambda b, r: (b, r, 0)),
        out_shape=jax.ShapeDtypeStruct((B, N, D), jnp.float32),
        scratch_shapes=[pltpu.VMEM((1, N), jnp.float32)],
    )(x, t.reshape(B, 1, 1), c_vector.reshape(B, 1, CTX), W_msg, b_msg,
      W_time, W_ctx, W_out, b_out)
    return out


# speedup vs baseline: 3.0617x; 1.0733x over previous
import functools
import math

import jax
import jax.numpy as jnp
from jax import lax
from jax.experimental import pallas as pl
from jax.experimental.pallas import tpu as pltpu

K = 16
R = 256
SUB = 2
C = 128


def _fused_kernel(x_ref, t_ref, c_ref, wmsg_ref, bmsg_ref, wtime_ref,
                  wctx_ref, wout_ref, bout_ref, out_ref, sq_ref):
    r = pl.program_id(1)
    xs = x_ref[0]
    n_nodes = xs.shape[0]

    @pl.when(r == 0)
    def _():
        sq_ref[...] = jnp.sum(xs * xs, axis=1)[None, :]

    half = wtime_ref.shape[0] // 2
    freq_i = jax.lax.broadcasted_iota(jnp.int32, (1, half), 1).astype(
        jnp.float32)
    freqs = jnp.exp(freq_i * (-math.log(10000.0) / half))
    args = t_ref[0] * freqs
    temb = jnp.concatenate([jnp.cos(args), jnp.sin(args)], axis=-1)
    cond = (jnp.dot(temb, wtime_ref[...], preferred_element_type=jnp.float32)
            + jnp.dot(c_ref[0], wctx_ref[...],
                      preferred_element_type=jnp.float32))

    for sub in range(SUB):
        rr = r * SUB + sub
        xr = x_ref[0, pl.ds(rr * R, R), :]
        cross = jax.lax.dot_general(
            xr, xs, (((1,), (1,)), ((), ())),
            preferred_element_type=jnp.float32)
        s = sq_ref[...] - 2.0 * cross
        rows = jax.lax.broadcasted_iota(jnp.int32, (R, n_nodes), 0) + rr * R
        cols = jax.lax.broadcasted_iota(jnp.int32, (R, n_nodes), 1)
        s = jnp.where(rows == cols, s + 1e9, s)

        nchunk = n_nodes // C
        inf = jnp.full((R, C), jnp.inf, dtype=jnp.float32)
        m1, m2, m3, m4 = inf, inf, inf, inf
        for g in range(nchunk):
            v = s[:, g * C:(g + 1) * C]
            b1 = jnp.maximum(m1, v)
            m1 = jnp.minimum(m1, v)
            b2 = jnp.maximum(m2, b1)
            m2 = jnp.minimum(m2, b1)
            b3 = jnp.maximum(m3, b2)
            m3 = jnp.minimum(m3, b2)
            m4 = jnp.minimum(m4, b3)
        tau = None
        for _ in range(K):
            tau = jnp.min(m1, axis=1, keepdims=True)
            eq = m1 == tau
            m1 = jnp.where(eq, m2, m1)
            m2 = jnp.where(eq, m3, m2)
            m3 = jnp.where(eq, m4, m3)
            m4 = jnp.where(eq, jnp.inf, m4)
        selmask = jnp.where(s <= tau, 1.0, 0.0)
        cnt = jnp.sum(selmask, axis=1, keepdims=True)
        nbad = jnp.sum(jnp.where(cnt == float(K), 0.0, 1.0))

        def _exact_mask(s=s):
            w = s
            m = None
            for _ in range(K):
                m = jnp.min(w, axis=1, keepdims=True)
                w = jnp.where(w == m, jnp.inf, w)
            return jnp.where(s <= m, 1.0, 0.0)

        mask = lax.cond(nbad > 0.0, _exact_mask,
                        lambda selmask=selmask: selmask)

        nbr = jnp.dot(mask, xs, preferred_element_type=jnp.float32)
        agg = (jnp.dot(nbr, wmsg_ref[...], preferred_element_type=jnp.float32)
               + float(K) * bmsg_ref[...][None, :])
        h = jnp.maximum(xr + agg + cond, 0.0)
        out_ref[0, sub * R:(sub + 1) * R, :] = (
            jnp.dot(h, wout_ref[...], preferred_element_type=jnp.float32)
            + bout_ref[...][None, :])


def kernel(x, t, c_vector, W_msg, b_msg, W_time, W_ctx, W_out, b_out):
    B, N, D = x.shape
    CTX = c_vector.shape[1]
    nb = N // (R * SUB)
    grid = (B, nb)
    out = pl.pallas_call(
        _fused_kernel,
        grid=grid,
        in_specs=[
            pl.BlockSpec((1, N, D), lambda b, r: (b, 0, 0)),
            pl.BlockSpec((1, 1, 1), lambda b, r: (b, 0, 0)),
            pl.BlockSpec((1, 1, CTX), lambda b, r: (b, 0, 0)),
            pl.BlockSpec((D, D), lambda b, r: (0, 0)),
            pl.BlockSpec((D,), lambda b, r: (0,)),
            pl.BlockSpec((D, D), lambda b, r: (0, 0)),
            pl.BlockSpec((CTX, D), lambda b, r: (0, 0)),
            pl.BlockSpec((D, D), lambda b, r: (0, 0)),
            pl.BlockSpec((D,), lambda b, r: (0,)),
        ],
        out_specs=pl.BlockSpec((1, R * SUB, D), lambda b, r: (b, r, 0)),
        out_shape=jax.ShapeDtypeStruct((B, N, D), jnp.float32),
        scratch_shapes=[pltpu.VMEM((1, N), jnp.float32)],
    )(x, t.reshape(B, 1, 1), c_vector.reshape(B, 1, CTX), W_msg, b_msg,
      W_time, W_ctx, W_out, b_out)
    return out


# speedup vs baseline: 3.1277x; 1.0216x over previous
import functools
import math

import jax
import jax.numpy as jnp
from jax import lax
from jax.experimental import pallas as pl
from jax.experimental.pallas import tpu as pltpu

K = 16
R = 256
SUB = 4
C = 128


def _fused_kernel(x_ref, t_ref, c_ref, wmsg_ref, bmsg_ref, wtime_ref,
                  wctx_ref, wout_ref, bout_ref, out_ref, sq_ref):
    r = pl.program_id(1)
    xs = x_ref[0]
    n_nodes = xs.shape[0]

    @pl.when(r == 0)
    def _():
        sq_ref[...] = jnp.sum(xs * xs, axis=1)[None, :]

    half = wtime_ref.shape[0] // 2
    freq_i = jax.lax.broadcasted_iota(jnp.int32, (1, half), 1).astype(
        jnp.float32)
    freqs = jnp.exp(freq_i * (-math.log(10000.0) / half))
    args = t_ref[0] * freqs
    temb = jnp.concatenate([jnp.cos(args), jnp.sin(args)], axis=-1)
    cond = (jnp.dot(temb, wtime_ref[...], preferred_element_type=jnp.float32)
            + jnp.dot(c_ref[0], wctx_ref[...],
                      preferred_element_type=jnp.float32))

    for sub in range(SUB):
        rr = r * SUB + sub
        xr = x_ref[0, pl.ds(rr * R, R), :]
        cross = jax.lax.dot_general(
            xr, xs, (((1,), (1,)), ((), ())),
            preferred_element_type=jnp.float32)
        s = sq_ref[...] - 2.0 * cross
        rows = jax.lax.broadcasted_iota(jnp.int32, (R, n_nodes), 0) + rr * R
        cols = jax.lax.broadcasted_iota(jnp.int32, (R, n_nodes), 1)
        s = jnp.where(rows == cols, s + 1e9, s)

        nchunk = n_nodes // C
        inf = jnp.full((R, C), jnp.inf, dtype=jnp.float32)
        m1, m2, m3, m4 = inf, inf, inf, inf
        for g in range(nchunk):
            v = s[:, g * C:(g + 1) * C]
            b1 = jnp.maximum(m1, v)
            m1 = jnp.minimum(m1, v)
            b2 = jnp.maximum(m2, b1)
            m2 = jnp.minimum(m2, b1)
            b3 = jnp.maximum(m3, b2)
            m3 = jnp.minimum(m3, b2)
            m4 = jnp.minimum(m4, b3)
        tau = None
        for _ in range(K):
            tau = jnp.min(m1, axis=1, keepdims=True)
            eq = m1 == tau
            m1 = jnp.where(eq, m2, m1)
            m2 = jnp.where(eq, m3, m2)
            m3 = jnp.where(eq, m4, m3)
            m4 = jnp.where(eq, jnp.inf, m4)
        selmask = jnp.where(s <= tau, 1.0, 0.0)
        cnt = jnp.sum(selmask, axis=1, keepdims=True)
        nbad = jnp.sum(jnp.where(cnt == float(K), 0.0, 1.0))

        def _exact_mask(s=s):
            w = s
            m = None
            for _ in range(K):
                m = jnp.min(w, axis=1, keepdims=True)
                w = jnp.where(w == m, jnp.inf, w)
            return jnp.where(s <= m, 1.0, 0.0)

        mask = lax.cond(nbad > 0.0, _exact_mask,
                        lambda selmask=selmask: selmask)

        nbr = jnp.dot(mask, xs, preferred_element_type=jnp.float32)
        agg = (jnp.dot(nbr, wmsg_ref[...], preferred_element_type=jnp.float32)
               + float(K) * bmsg_ref[...][None, :])
        h = jnp.maximum(xr + agg + cond, 0.0)
        out_ref[0, sub * R:(sub + 1) * R, :] = (
            jnp.dot(h, wout_ref[...], preferred_element_type=jnp.float32)
            + bout_ref[...][None, :])


def kernel(x, t, c_vector, W_msg, b_msg, W_time, W_ctx, W_out, b_out):
    B, N, D = x.shape
    CTX = c_vector.shape[1]
    nb = N // (R * SUB)
    grid = (B, nb)
    out = pl.pallas_call(
        _fused_kernel,
        grid=grid,
        in_specs=[
            pl.BlockSpec((1, N, D), lambda b, r: (b, 0, 0)),
            pl.BlockSpec((1, 1, 1), lambda b, r: (b, 0, 0)),
            pl.BlockSpec((1, 1, CTX), lambda b, r: (b, 0, 0)),
            pl.BlockSpec((D, D), lambda b, r: (0, 0)),
            pl.BlockSpec((D,), lambda b, r: (0,)),
            pl.BlockSpec((D, D), lambda b, r: (0, 0)),
            pl.BlockSpec((CTX, D), lambda b, r: (0, 0)),
            pl.BlockSpec((D, D), lambda b, r: (0, 0)),
            pl.BlockSpec((D,), lambda b, r: (0,)),
        ],
        out_specs=pl.BlockSpec((1, R * SUB, D), lambda b, r: (b, r, 0)),
        out_shape=jax.ShapeDtypeStruct((B, N, D), jnp.float32),
        scratch_shapes=[pltpu.VMEM((1, N), jnp.float32)],
    )(x, t.reshape(B, 1, 1), c_vector.reshape(B, 1, CTX), W_msg, b_msg,
      W_time, W_ctx, W_out, b_out)
    return out
